# Initial kernel scaffold; baseline (speedup 1.0000x reference)
import functools

import jax
import jax.numpy as jnp
from jax import lax
from jax.experimental import pallas as pl
from jax.experimental.pallas import tpu as pltpu
from jax.experimental.pallas import tpu_sc as plsc

B = 16384
L = 20
EMB = 32
NC = 2
NS = 16
NW = NC * NS
BPW = B // NW
C = 128
NCHUNK = BPW // C


def _make_kernel():
    mesh = plsc.VectorSubcoreMesh(core_axis_name="c", subcore_axis_name="s")

    @functools.partial(
        pl.kernel,
        mesh=mesh,
        out_type=jax.ShapeDtypeStruct((B, 2 * EMB), jnp.float32),
        scratch_types=[
            pltpu.VMEM((L, C), jnp.int32),
            pltpu.VMEM((L, C, EMB), jnp.float32),
            pltpu.VMEM((C,), jnp.int32),
            pltpu.VMEM((C, EMB), jnp.float32),
            pltpu.VMEM((C, 2 * EMB), jnp.float32),
            pltpu.VMEM((C,), jnp.float32),
            pltpu.VMEM((C,), jnp.float32),
            pltpu.VMEM((1, EMB), jnp.float32),
            pltpu.SemaphoreType.DMA,
        ],
    )
    def kern(title_idx_h, tok_t_h, title_tab_h, text_tab_h, out_h,
             ids_v, rows_v, tidx_v, trows_v, outv, inv_v, s2_v, t0_v, sem):
        wid = lax.axis_index("s") * NC + lax.axis_index("c")

        pltpu.sync_copy(text_tab_h.at[pl.ds(0, 1)], t0_v)
        t0a = t0_v[0, pl.ds(0, 16)]
        t0b = t0_v[0, pl.ds(16, 16)]

        for ci in range(NCHUNK):
            base = wid * BPW + ci * C

            for l in range(L):
                pltpu.sync_copy(tok_t_h.at[l, pl.ds(base, C)], ids_v.at[l])
            pltpu.sync_copy(title_idx_h.at[pl.ds(base, C)], tidx_v)

            cps = [pltpu.async_copy(title_tab_h.at[tidx_v], trows_v, sem)]
            for l in range(L):
                cps.append(pltpu.async_copy(
                    text_tab_h.at[ids_v.at[l]], rows_v.at[l], sem))

            def count_body(g, carry):
                g16 = g * 16
                n = jnp.zeros((16,), jnp.float32)
                for l in range(L):
                    idv = ids_v[l, pl.ds(g16, 16)]
                    n = n + jnp.where(idv != 0, jnp.float32(1), jnp.float32(0))
                inv = jnp.float32(1) / jnp.maximum(n, jnp.float32(1))
                inv_v[pl.ds(g16, 16)] = inv
                s2_v[pl.ds(g16, 16)] = (jnp.float32(L) - n) * inv
                return carry
            lax.fori_loop(0, C // 16, count_body, 0)

            for cp in cps:
                cp.wait()

            def row_body(b, carry):
                lane = b & 15
                goff = b - lane
                lanes = jnp.full((16,), lane, dtype=jnp.int32)
                s1 = jnp.take(inv_v[pl.ds(goff, 16)], lanes, axis=0,
                              mode="promise_in_bounds")
                s2 = jnp.take(s2_v[pl.ds(goff, 16)], lanes, axis=0,
                              mode="promise_in_bounds")
                for j in range(2):
                    js = pl.ds(j * 16, 16)
                    acc = rows_v[0, b, js]
                    for l in range(1, L):
                        acc = acc + rows_v[l, b, js]
                    t0j = t0a if j == 0 else t0b
                    outv[b, pl.ds(j * 16, 16)] = trows_v[b, js]
                    outv[b, pl.ds(EMB + j * 16, 16)] = acc * s1 - s2 * t0j
                return carry
            lax.fori_loop(0, C, row_body, 0)

            pltpu.sync_copy(outv, out_h.at[pl.ds(base, C)])

    return kern


_kern = _make_kernel()


@jax.jit
def kernel(title_idx, token_ids, title_table, text_table):
    tok_t = token_ids.T
    return _kern(title_idx, tok_t, title_table, text_table)


# speedup vs baseline: 10.5721x; 10.5721x over previous
import functools

import jax
import jax.numpy as jnp
from jax import lax
from jax.experimental import pallas as pl
from jax.experimental.pallas import tpu as pltpu
from jax.experimental.pallas import tpu_sc as plsc

B = 16384
L = 20
EMB = 32
NC = 2
NS = 16
NW = NC * NS
BPW = B // NW
C = 128
NCHUNK = BPW // C


def _splat(vec, lanes):
    dnums = lax.GatherDimensionNumbers(
        offset_dims=(), collapsed_slice_dims=(0,), start_index_map=(0,))
    return lax.gather(vec, lanes[:, None], dnums, slice_sizes=(1,),
                      mode=lax.GatherScatterMode.PROMISE_IN_BOUNDS)


def _make_kernel():
    mesh = plsc.VectorSubcoreMesh(core_axis_name="c", subcore_axis_name="s")

    @functools.partial(
        pl.kernel,
        mesh=mesh,
        out_type=jax.ShapeDtypeStruct((B, 2 * EMB), jnp.float32),
        scratch_types=[
            pltpu.VMEM((L, C), jnp.int32),
            pltpu.VMEM((L, C, EMB), jnp.float32),
            pltpu.VMEM((C,), jnp.int32),
            pltpu.VMEM((C, EMB), jnp.float32),
            pltpu.VMEM((C, 2 * EMB), jnp.float32),
            pltpu.VMEM((C,), jnp.float32),
            pltpu.VMEM((C,), jnp.float32),
            pltpu.VMEM((1, EMB), jnp.float32),
            pltpu.SemaphoreType.DMA,
        ],
        compiler_params=pltpu.CompilerParams(use_tc_tiling_on_sc=False),
    )
    def kern(title_idx_h, tok_t_h, title_tab_h, text_tab_h, out_h,
             ids_v, rows_v, tidx_v, trows_v, outv, inv_v, s2_v, t0_v, sem):
        wid = lax.axis_index("s") * NC + lax.axis_index("c")

        pltpu.sync_copy(text_tab_h.at[pl.ds(0, 1)], t0_v)
        t0a = t0_v[0, pl.ds(0, 16)]
        t0b = t0_v[0, pl.ds(16, 16)]

        for ci in range(NCHUNK):
            base = wid * BPW + ci * C

            for l in range(L):
                pltpu.sync_copy(tok_t_h.at[l, pl.ds(base, C)], ids_v.at[l])
            pltpu.sync_copy(title_idx_h.at[pl.ds(base, C)], tidx_v)

            cps = [pltpu.async_copy(title_tab_h.at[tidx_v], trows_v, sem)]
            for l in range(L):
                cps.append(pltpu.async_copy(
                    text_tab_h.at[ids_v.at[l]], rows_v.at[l], sem))

            def count_body(g, carry):
                g16 = g * 16
                n = jnp.zeros((16,), jnp.float32)
                for l in range(L):
                    idv = ids_v[l, pl.ds(g16, 16)]
                    n = n + jnp.where(idv != 0, jnp.float32(1), jnp.float32(0))
                inv = jnp.float32(1) / jnp.maximum(n, jnp.float32(1))
                inv_v[pl.ds(g16, 16)] = inv
                s2_v[pl.ds(g16, 16)] = (jnp.float32(L) - n) * inv
                return carry
            lax.fori_loop(0, C // 16, count_body, 0)

            for cp in cps:
                cp.wait()

            def row_body(b, carry):
                lane = b & 15
                goff = b - lane
                lanes = jnp.full((16,), lane, dtype=jnp.int32)
                s1 = _splat(inv_v[pl.ds(goff, 16)], lanes)
                s2 = _splat(s2_v[pl.ds(goff, 16)], lanes)
                for j in range(2):
                    js = pl.ds(j * 16, 16)
                    acc = rows_v[0, b, js]
                    for l in range(1, L):
                        acc = acc + rows_v[l, b, js]
                    t0j = t0a if j == 0 else t0b
                    outv[b, pl.ds(j * 16, 16)] = trows_v[b, js]
                    outv[b, pl.ds(EMB + j * 16, 16)] = acc * s1 - s2 * t0j
                return carry
            lax.fori_loop(0, C, row_body, 0)

            pltpu.sync_copy(outv, out_h.at[pl.ds(base, C)])

    return kern


_kern = _make_kernel()


@jax.jit
def kernel(title_idx, token_ids, title_table, text_table):
    tok_t = token_ids.T
    return _kern(title_idx, tok_t, title_table, text_table)


# speedup vs baseline: 16.3774x; 1.5491x over previous
import functools

import jax
import jax.numpy as jnp
from jax import lax
from jax.experimental import pallas as pl
from jax.experimental.pallas import tpu as pltpu
from jax.experimental.pallas import tpu_sc as plsc

B = 16384
L = 20
EMB = 32
NC = 2
NS = 16
NW = NC * NS
BPW = B // NW
C = 64
NCHUNK = BPW // C


def _splat(vec, lane):
    lanes = jnp.full((16,), lane, dtype=jnp.int32)
    dnums = lax.GatherDimensionNumbers(
        offset_dims=(), collapsed_slice_dims=(0,), start_index_map=(0,))
    return lax.gather(vec, lanes[:, None], dnums, slice_sizes=(1,),
                      mode=lax.GatherScatterMode.PROMISE_IN_BOUNDS)


def _tree_sum(vals):
    vals = list(vals)
    while len(vals) > 1:
        nxt = [a + b for a, b in zip(vals[::2], vals[1::2])]
        if len(vals) % 2:
            nxt.append(vals[-1])
        vals = nxt
    return vals[0]


def _make_kernel():
    mesh = plsc.VectorSubcoreMesh(core_axis_name="c", subcore_axis_name="s")

    @functools.partial(
        pl.kernel,
        mesh=mesh,
        out_type=jax.ShapeDtypeStruct((B, 2 * EMB), jnp.float32),
        scratch_types=[
            pltpu.VMEM((L * NCHUNK, C), jnp.int32),
            pltpu.VMEM((NCHUNK, C), jnp.int32),
            pltpu.VMEM((2, L, C, EMB), jnp.float32),
            pltpu.VMEM((2, C, EMB), jnp.float32),
            pltpu.VMEM((2, C, 2 * EMB), jnp.float32),
            pltpu.VMEM((BPW,), jnp.float32),
            pltpu.VMEM((BPW,), jnp.float32),
            pltpu.VMEM((1, EMB), jnp.float32),
            [pltpu.SemaphoreType.DMA] * 2,
            pltpu.SemaphoreType.DMA,
            pltpu.SemaphoreType.DMA,
        ],
        compiler_params=pltpu.CompilerParams(use_tc_tiling_on_sc=False),
    )
    def kern(t2_h, tok3_h, title_tab_h, text_tab_h, out_h,
             ids_v, tidx_v, rows_v, trows_v, outv, inv_v, s2_v, t0_v,
             gsems, ssem, osem):
        wid = lax.axis_index("s") * NC + lax.axis_index("c")
        base_w = wid * BPW

        stage = [pltpu.async_copy(
            tok3_h.at[l, pl.ds(wid * NCHUNK, NCHUNK)],
            ids_v.at[pl.ds(l * NCHUNK, NCHUNK)], ssem) for l in range(L)]
        stage.append(pltpu.async_copy(
            t2_h.at[pl.ds(wid * NCHUNK, NCHUNK)], tidx_v, ssem))
        stage.append(pltpu.async_copy(text_tab_h.at[pl.ds(0, 1)], t0_v, ssem))
        for cp in stage:
            cp.wait()
        t0a = t0_v[0, pl.ds(0, 16)]
        t0b = t0_v[0, pl.ds(16, 16)]

        def fire(ci):
            buf = ci % 2
            cps = [pltpu.async_copy(
                title_tab_h.at[tidx_v.at[ci]], trows_v.at[buf], gsems[buf])]
            for l in range(L):
                cps.append(pltpu.async_copy(
                    text_tab_h.at[ids_v.at[l * NCHUNK + ci]],
                    rows_v.at[buf, l], gsems[buf]))
            return cps

        inflight = fire(0)

        def count_body(g, carry):
            ci = g >> 2
            off = (g & 3) * 16
            n = jnp.zeros((16,), jnp.float32)
            for l in range(L):
                idv = ids_v[l * NCHUNK + ci, pl.ds(off, 16)]
                n = n + jnp.where(idv != 0, jnp.float32(1), jnp.float32(0))
            inv = jnp.float32(1) / jnp.maximum(n, jnp.float32(1))
            base = ci * C + off
            inv_v[pl.ds(base, 16)] = inv
            s2_v[pl.ds(base, 16)] = (jnp.float32(L) - n) * inv
            return carry
        lax.fori_loop(0, BPW // 16, count_body, 0)

        out_cps = [None, None]
        for ci in range(NCHUNK):
            buf = ci % 2
            nxt = inflight if ci + 1 == NCHUNK else fire(ci + 1)
            for cp in inflight:
                cp.wait()
            inflight = nxt

            if out_cps[buf] is not None:
                out_cps[buf].wait()

            @plsc.parallel_loop(0, C)
            def row_body(b):
                r = ci * C + b
                lane = r & 15
                goff = r - lane
                s1 = _splat(inv_v[pl.ds(goff, 16)], lane)
                s2 = _splat(s2_v[pl.ds(goff, 16)], lane)
                for j in range(2):
                    js = pl.ds(j * 16, 16)
                    acc = _tree_sum(
                        rows_v[buf, l, b, js] for l in range(L))
                    t0j = t0a if j == 0 else t0b
                    outv[buf, b, pl.ds(j * 16, 16)] = trows_v[buf, b, js]
                    outv[buf, b, pl.ds(EMB + j * 16, 16)] = acc * s1 - s2 * t0j

            out_cps[buf] = pltpu.async_copy(
                outv.at[buf], out_h.at[pl.ds(base_w + ci * C, C)], osem)

        for cp in out_cps:
            if cp is not None:
                cp.wait()

    return kern


_kern = _make_kernel()


@jax.jit
def kernel(title_idx, token_ids, title_table, text_table):
    tok3 = token_ids.T.reshape(L, B // C, C)
    t2 = title_idx.reshape(B // C, C)
    return _kern(t2, tok3, title_table, text_table)
